# baseline (speedup 1.0000x reference)
import functools

import jax
import jax.numpy as jnp
from jax import lax
from jax.experimental import pallas as pl
from jax.experimental.pallas import tpu as pltpu
from jax.experimental.pallas import tpu_sc as plsc

VOCAB = 1000
VPAD = 1024
N_TOK = 1024 * 50
NC, NS, L = 2, 16, 16
NW = NC * NS
TPW = N_TOK // NW
CH = 32
NCH = TPW // CH
RPC = CH * 8
PROWS = VOCAB * 8
OROWS = N_TOK * 8
LROWS = (TPW // L + 7) // 8


def _prep_body(table_ref, lse_ref, pad_ref):
    x = table_ref[...]
    m = jnp.max(x, axis=1, keepdims=True)
    s = jnp.sum(jnp.exp(x - m), axis=1, keepdims=True)
    lse_ref[...] = jnp.log(s) + m
    for q in range(7):
        pad_ref[:, q, :] = x[:, 128 * q:128 * (q + 1)]
    pad_ref[:, 7, :] = jnp.concatenate(
        [x[:, 896:], jnp.zeros((VOCAB, VPAD - VOCAB), jnp.float32)], axis=1)


_prep_call = pl.pallas_call(
    _prep_body,
    out_shape=[
        jax.ShapeDtypeStruct((VOCAB, 1), jnp.float32),
        jax.ShapeDtypeStruct((VOCAB, 8, 128), jnp.float32),
    ],
)


def _sc_body(pad_hbm, idx_hbm, tgt_hbm, lse_hbm, out_hbm, psum_hbm,
             idx_v, tgt_v, lse_v, rows0, rows1, gbuf, acc_v,
             gsem0, gsem1, osem0, osem1):
    wid = lax.axis_index("s") * NC + lax.axis_index("c")
    base = wid * TPW
    obase = base * 8
    rows = (rows0, rows1)
    gsem = (gsem0, gsem1)
    osem = (osem0, osem1)

    pltpu.sync_copy(idx_hbm.at[pl.ds(base, TPW)], idx_v)
    pltpu.sync_copy(tgt_hbm.at[pl.ds(base, TPW)], tgt_v)
    pltpu.sync_copy(lse_hbm, lse_v)

    def fill_gidx(c, b):
        j16 = lax.iota(jnp.int32, L)
        for s in range(16):
            m16 = j16 + 16 * s
            kvec = 8 * (m16 // 64) + m16 % 8 + c * CH
            ivec = plsc.load_gather(idx_v, [kvec])
            gidx = ivec * 8 + (m16 // 8) % 8
            gbuf[b, s // 8, pl.ds((s % 8) * L, L)] = gidx

    def issue_gather(b):
        for i in range(2):
            pltpu.async_copy(pad_hbm.at[gbuf.at[b, i]],
                             rows[b].at[pl.ds(128 * i, 128)], gsem[b])

    def wait_gather(b):
        for i in range(2):
            pltpu.make_async_copy(pad_hbm.at[gbuf.at[b, 0]],
                                  rows[b].at[pl.ds(0, 128)], gsem[b]).wait()

    def issue_out(c, b):
        pltpu.async_copy(rows[b], out_hbm.at[pl.ds(obase + RPC * c, RPC)],
                         osem[b])

    def wait_out(b):
        pltpu.make_async_copy(rows[b], out_hbm.at[pl.ds(obase, RPC)],
                              osem[b]).wait()

    def extract(c, b):
        j16 = lax.iota(jnp.int32, L)
        for h in range(CH // L):
            kk = j16 + 16 * h
            tgt16 = tgt_v[pl.ds(c * CH + 16 * h, L)]
            ridx = 64 * (kk // 8) + 8 * (tgt16 // 128) + kk % 8
            val16 = plsc.load_gather(rows[b], [ridx, tgt16 % 128])
            acc_v[...] = acc_v[...] - val16

    fill_gidx(0, 0)
    issue_gather(0)

    acc_v[...] = jnp.zeros((L,), jnp.float32)
    for c in range(TPW // L):
        i16 = idx_v[pl.ds(c * L, L)]
        acc_v[...] = acc_v[...] + plsc.load_gather(lse_v, [i16])

    def pair_body(s2, carry):
        for par in range(2):
            b = par
            nb = 1 - par
            c = 2 * s2 + par

            @pl.when(c < NCH - 1)
            def _(c=c, b=b, nb=nb):
                @pl.when(c >= 1)
                def _():
                    wait_out(nb)

                fill_gidx(c + 1, nb)
                issue_gather(nb)

            wait_gather(b)
            issue_out(c, b)
            extract(c, b)
        return carry

    lax.fori_loop(0, NCH // 2, pair_body, 0)
    wait_out(0)
    wait_out(1)
    pltpu.sync_copy(acc_v, psum_hbm.at[wid])


_sc_call = functools.partial(
    pl.kernel,
    mesh=plsc.VectorSubcoreMesh(core_axis_name="c", subcore_axis_name="s"),
    compiler_params=pltpu.CompilerParams(
        use_tc_tiling_on_sc=False, needs_layout_passes=False),
    out_type=[
        jax.ShapeDtypeStruct((OROWS, 128), jnp.float32),
        jax.ShapeDtypeStruct((NW, L), jnp.float32),
    ],
    scratch_types=[
        pltpu.VMEM((TPW,), jnp.int32),
        pltpu.VMEM((TPW,), jnp.int32),
        pltpu.VMEM((VOCAB,), jnp.float32),
        pltpu.VMEM((RPC, 128), jnp.float32),
        pltpu.VMEM((RPC, 128), jnp.float32),
        pltpu.VMEM((2, 2, 128), jnp.int32),
        pltpu.VMEM((L,), jnp.float32),
        pltpu.SemaphoreType.DMA,
        pltpu.SemaphoreType.DMA,
        pltpu.SemaphoreType.DMA,
        pltpu.SemaphoreType.DMA,
    ],
)(_sc_body)


def _loss_body(psum_ref, out_ref):
    out_ref[...] = jnp.sum(psum_ref[...], keepdims=True) / N_TOK


_loss_call = pl.pallas_call(
    _loss_body,
    out_shape=jax.ShapeDtypeStruct((1, 1), jnp.float32),
)


def kernel(idx, target, embedding_table):
    idxf = idx.reshape(-1).astype(jnp.int32)
    tgtf = target.reshape(-1).astype(jnp.int32)
    table = embedding_table.astype(jnp.float32)
    lse2, pad3 = _prep_call(table)
    lse = lse2.reshape(VOCAB)
    pad2d = pad3.reshape(PROWS, 128)
    out2d, psum = _sc_call(pad2d, idxf, tgtf, lse)
    logits = (out2d.reshape(N_TOK // 8, 8, 8, 128)
              .transpose(0, 2, 1, 3)
              .reshape(N_TOK, VPAD)[:, :VOCAB])
    loss = _loss_call(psum).reshape(())
    return logits, loss


# speedup vs baseline: 2.1755x; 2.1755x over previous
import functools

import jax
import jax.numpy as jnp
from jax import lax
from jax.experimental import pallas as pl
from jax.experimental.pallas import tpu as pltpu
from jax.experimental.pallas import tpu_sc as plsc

VOCAB = 1000
N_TOK = 1024 * 50
NC, NS, L = 2, 16, 16
NW = NC * NS
NSTRIPE = VOCAB // 8
SPW = 4
BLK = 2048
NBLK = N_TOK // BLK
TILES = BLK // 128
TPW = N_TOK // NW
LROWS = (TPW + 127) // 128


def _lse_body(table_ref, lse_ref):
    x = table_ref[...]
    m = jnp.max(x, axis=1, keepdims=True)
    s = jnp.sum(jnp.exp(x - m), axis=1, keepdims=True)
    lse_ref[...] = jnp.log(s) + m


_lse_call = pl.pallas_call(
    _lse_body,
    out_shape=jax.ShapeDtypeStruct((VOCAB, 1), jnp.float32),
)


def _sc_body(tflat_hbm, idx_hbm, tgt_hbm, lse_hbm, out_hbm, psum_hbm,
             tb0, tb1, tb2, tb3, sg0, sg1, sg2, sg3,
             idxb_v, lse_v, idxo_v, tgto_v, offs_v, vals_v, acc_v,
             dsem0, dsem1, dsem2, dsem3, strsem):
    wid = lax.axis_index("s") * NC + lax.axis_index("c")
    tblk = (tb0, tb1, tb2, tb3)
    stg = (sg0, sg1, sg2, sg3)
    dsem = (dsem0, dsem1, dsem2, dsem3)

    pltpu.sync_copy(lse_hbm, lse_v)
    pltpu.sync_copy(idx_hbm.at[pl.ds(wid * TPW, TPW)], idxo_v)
    pltpu.sync_copy(tgt_hbm.at[pl.ds(wid * TPW, TPW)], tgto_v)
    acc_v[...] = jnp.zeros((L,), jnp.float32)
    for c in range(TPW // L):
        sl = pl.ds(c * L, L)
        i16 = idxo_v[sl]
        offs_v[c // 8, pl.ds((c % 8) * L, L)] = tgto_v[sl] * VOCAB + i16
        acc_v[...] = acc_v[...] + plsc.load_gather(lse_v, [i16])
    for c in range(LROWS * 8 - TPW // L):
        offs_v[LROWS - 1, pl.ds((TPW // L % 8 + c) * L, L)] = (
            jnp.zeros((L,), jnp.int32))
    for r in range(LROWS):
        pltpu.async_copy(tflat_hbm.at[offs_v.at[r]], vals_v.at[r], strsem)
    for r in range(LROWS):
        pltpu.make_async_copy(
            tflat_hbm.at[offs_v.at[0]], vals_v.at[0], strsem).wait()
    for c in range(TPW // L):
        acc_v[...] = acc_v[...] - vals_v[c // 8, pl.ds((c % 8) * L, L)]
    pltpu.sync_copy(acc_v, psum_hbm.at[wid])

    for j in range(SPW):
        sj = wid + NW * j

        @pl.when(sj < NSTRIPE)
        def _(j=j, sj=sj):
            pltpu.sync_copy(tflat_hbm.at[pl.ds(sj * 8 * VOCAB, 8 * VOCAB)],
                            tblk[j])

    def blk_body(b, carry):
        pltpu.sync_copy(idx_hbm.at[pl.ds(b * BLK, BLK)], idxb_v)
        for j in range(SPW):
            sj = wid + NW * j

            @pl.when(sj < NSTRIPE)
            def _(j=j, sj=sj):
                @pl.when(b > 0)
                def _():
                    pltpu.make_async_copy(
                        stg[j], out_hbm.at[sj, pl.ds(0, TILES)],
                        dsem[j]).wait()

                @plsc.parallel_loop(0, TILES, 1, unroll=2)
                def tile_body(t, j=j):
                    for ch in range(8):
                        i16 = idxb_v[pl.ds(t * 128 + ch * L, L)]
                        for cc in range(8):
                            v16 = plsc.load_gather(tblk[j], [i16 + cc * VOCAB])
                            stg[j][t, cc, pl.ds(ch * L, L)] = v16
                pltpu.async_copy(
                    stg[j], out_hbm.at[sj, pl.ds(b * TILES, TILES)], dsem[j])
        return carry

    lax.fori_loop(0, NBLK, blk_body, 0)
    for j in range(SPW):
        sj = wid + NW * j

        @pl.when(sj < NSTRIPE)
        def _(j=j, sj=sj):
            pltpu.make_async_copy(
                stg[j], out_hbm.at[sj, pl.ds(0, TILES)], dsem[j]).wait()


_sc_call = functools.partial(
    pl.kernel,
    mesh=plsc.VectorSubcoreMesh(core_axis_name="c", subcore_axis_name="s"),
    compiler_params=pltpu.CompilerParams(
        use_tc_tiling_on_sc=False, needs_layout_passes=False),
    out_type=[
        jax.ShapeDtypeStruct((NSTRIPE, N_TOK // 128, 8, 128), jnp.float32),
        jax.ShapeDtypeStruct((NW, L), jnp.float32),
    ],
    scratch_types=[
        pltpu.VMEM((8 * VOCAB,), jnp.float32),
        pltpu.VMEM((8 * VOCAB,), jnp.float32),
        pltpu.VMEM((8 * VOCAB,), jnp.float32),
        pltpu.VMEM((8 * VOCAB,), jnp.float32),
        pltpu.VMEM((TILES, 8, 128), jnp.float32),
        pltpu.VMEM((TILES, 8, 128), jnp.float32),
        pltpu.VMEM((TILES, 8, 128), jnp.float32),
        pltpu.VMEM((TILES, 8, 128), jnp.float32),
        pltpu.VMEM((BLK,), jnp.int32),
        pltpu.VMEM((VOCAB,), jnp.float32),
        pltpu.VMEM((TPW,), jnp.int32),
        pltpu.VMEM((TPW,), jnp.int32),
        pltpu.VMEM((LROWS, 128), jnp.int32),
        pltpu.VMEM((LROWS, 128), jnp.float32),
        pltpu.VMEM((L,), jnp.float32),
        pltpu.SemaphoreType.DMA,
        pltpu.SemaphoreType.DMA,
        pltpu.SemaphoreType.DMA,
        pltpu.SemaphoreType.DMA,
        pltpu.SemaphoreType.DMA,
    ],
)(_sc_body)


def _loss_body(psum_ref, out_ref):
    out_ref[...] = jnp.sum(psum_ref[...], keepdims=True) / N_TOK


_loss_call = pl.pallas_call(
    _loss_body,
    out_shape=jax.ShapeDtypeStruct((1, 1), jnp.float32),
)


def kernel(idx, target, embedding_table):
    idxf = idx.reshape(-1).astype(jnp.int32)
    tgtf = target.reshape(-1).astype(jnp.int32)
    table = embedding_table.astype(jnp.float32)
    lse = _lse_call(table).reshape(VOCAB)
    tflat = table.T.reshape(-1)
    out4d, psum = _sc_call(tflat, idxf, tgtf, lse)
    logits = out4d.transpose(1, 3, 0, 2).reshape(N_TOK, VOCAB)
    loss = _loss_call(psum).reshape(())
    return logits, loss
